# baseline (speedup 1.0000x reference)
import functools

import jax
import jax.numpy as jnp
from jax import lax
from jax.experimental import pallas as pl
from jax.experimental.pallas import tpu as pltpu
from jax.experimental.pallas import tpu_sc as plsc

N = 10000
E = 320000
C_IN = 128
C_HID = 128
C_OUT = 64
FS = 32

NC = 2
NS = 16
NW = NC * NS

CH = 128
GRP = 5

EPT = 20480
NCH = EPT // CH
NGRP = NCH // GRP
PADT = EPT - E // NS

NCHD = NCH // NC
NT = N + 128

BR = 624
XR = N - NS * BR
ZR = 208

DEGW = 16

_f32 = jnp.float32


def _mesh():
    return plsc.VectorSubcoreMesh(core_axis_name="c", subcore_axis_name="s")


_sc_params = pltpu.CompilerParams(use_tc_tiling_on_sc=False)


def _fill_zero(ref, rows, cols):
    zero16 = jnp.zeros((16,), _f32)

    def body(i, _):
        for j in range(cols // 16):
            ref[i, pl.ds(j * 16, 16)] = zero16
        return 0
    lax.fori_loop(0, rows, body, 0)


def _zero_and_barrier(zbuf_v, acc_sh, sid):
    for k in range(BR // ZR):
        pltpu.sync_copy(zbuf_v, acc_sh.at[pl.ds(sid * BR + k * ZR, ZR)])

    @pl.when(sid == NS - 1)
    def _():
        pltpu.sync_copy(zbuf_v.at[pl.ds(0, XR)],
                        acc_sh.at[pl.ds(NS * BR, XR)])
    plsc.subcore_barrier()


def _copy_out(acc_sh, out_slice, sid):
    pltpu.sync_copy(acc_sh.at[pl.ds(sid * BR, BR)],
                    out_slice.at[pl.ds(sid * BR, BR)])

    @pl.when(sid == NS - 1)
    def _():
        pltpu.sync_copy(acc_sh.at[pl.ds(NS * BR, XR)],
                        out_slice.at[pl.ds(NS * BR, XR)])


@functools.partial(
    pl.kernel,
    out_type=jax.ShapeDtypeStruct((NC, N, DEGW), _f32),
    mesh=_mesh(),
    compiler_params=_sc_params,
    scratch_types=[
        pltpu.VMEM((NCHD, CH), jnp.int32),
        pltpu.VMEM((CH, DEGW), _f32),
        pltpu.VMEM((ZR, DEGW), _f32),
        pltpu.VMEM_SHARED((NT, DEGW), _f32),
        pltpu.SemaphoreType.DMA,
    ],
)
def _deg_kernel(col_hbm, out_hbm, col_v, ones_v, zbuf_v, acc_sh, sem):
    cid = lax.axis_index("c")
    sid = lax.axis_index("s")

    pltpu.sync_copy(col_hbm.at[sid, pl.ds(cid * NCHD, NCHD)], col_v)

    one16 = jnp.ones((16,), _f32)

    def fill_ones(i, _):
        ones_v[i, :] = one16
        return 0
    lax.fori_loop(0, CH, fill_ones, 0)

    _fill_zero(zbuf_v, ZR, DEGW)
    _zero_and_barrier(zbuf_v, acc_sh, sid)

    def fire(g):
        for b in range(GRP):
            pltpu.async_copy(ones_v, acc_sh.at[col_v.at[g * GRP + b]],
                             sem, add=True)

    def drain(g):
        for b in range(GRP):
            pltpu.make_async_copy(ones_v, acc_sh.at[col_v.at[g * GRP + b]],
                                  sem).wait()

    fire(0)

    def body(g, _):
        @pl.when(g + 1 < NCHD // GRP)
        def _():
            fire(g + 1)
        drain(g)
        return 0
    lax.fori_loop(0, NCHD // GRP, body, 0)

    plsc.subcore_barrier()
    _copy_out(acc_sh, out_hbm.at[cid], sid)


NBUF = 8
LAG = 2


def _make_scatter(NBLK):
    @functools.partial(
        pl.kernel,
        out_type=jax.ShapeDtypeStruct((NBLK, N, FS), _f32),
        mesh=_mesh(),
        compiler_params=_sc_params,
        scratch_types=[
            pltpu.VMEM((NCH, CH), jnp.int32),
            pltpu.VMEM((NCH, CH), jnp.int32),
            pltpu.VMEM((NBUF, CH, FS), _f32),
            pltpu.VMEM((ZR, FS), _f32),
            pltpu.VMEM_SHARED((NT, FS), _f32),
            pltpu.SemaphoreType.DMA,
            pltpu.SemaphoreType.DMA,
        ],
    )
    def _scatter_kernel(row_hbm, col_hbm, tab_hbm, out_hbm,
                        idx_v, col_v, buf_v, zbuf_v, acc_sh, semg, sems):
        cid = lax.axis_index("c")
        sid = lax.axis_index("s")

        pltpu.sync_copy(row_hbm.at[sid], idx_v)
        pltpu.sync_copy(col_hbm.at[sid], col_v)
        _fill_zero(zbuf_v, ZR, FS)

        for p in range(NBLK // NC):
            blk = NC * p + cid
            step16 = (jnp.zeros((16,), jnp.int32) +
                      (blk if p == 0 else NC))

            def mkidx(j, _):
                for l in range(CH // 16):
                    sl = pl.ds(l * 16, 16)
                    v = idx_v[j, sl]
                    idx_v[j, sl] = (v * NBLK if p == 0 else v) + step16
                return 0
            lax.fori_loop(0, NCH, mkidx, 0)

            _zero_and_barrier(zbuf_v, acc_sh, sid)

            def fire_gather(j):
                slot = lax.rem(j, NBUF)
                pltpu.async_copy(tab_hbm.at[idx_v.at[j]],
                                 buf_v.at[slot], semg)

            def wait_scatter(j):
                slot = lax.rem(j, NBUF)
                pltpu.make_async_copy(buf_v.at[slot],
                                      acc_sh.at[col_v.at[j]], sems).wait()

            def prologue(j, _):
                fire_gather(j)
                return 0
            lax.fori_loop(0, NBUF, prologue, 0)

            def chunk(j, _):
                slot = lax.rem(j, NBUF)

                @pl.when(j >= LAG)
                def _():
                    wait_scatter(j - LAG)

                @pl.when(jnp.logical_and(j >= LAG, j + NBUF - LAG < NCH))
                def _():
                    fire_gather(j - LAG + NBUF)

                pltpu.make_async_copy(tab_hbm.at[idx_v.at[j]],
                                      buf_v.at[slot], semg).wait()
                pltpu.async_copy(buf_v.at[slot], acc_sh.at[col_v.at[j]],
                                 sems, add=True)
                return 0
            lax.fori_loop(0, NCH, chunk, 0)

            def epilogue(j, _):
                wait_scatter(j)
                return 0
            lax.fori_loop(NCH - LAG, NCH, epilogue, 0)

            plsc.subcore_barrier()
            _copy_out(acc_sh, out_hbm.at[blk], sid)
            plsc.subcore_barrier()

    return _scatter_kernel


_scatter_hid = _make_scatter(C_HID // FS)
_scatter_out = _make_scatter(C_OUT // FS)


RB = 400
NRB = N // RB


def _tc_mm_body(x_ref, w1_ref, h_ref):
    h_ref[...] = jnp.dot(x_ref[...], w1_ref[...],
                         preferred_element_type=_f32)


@jax.jit
def _tc_mm(x, W1):
    return pl.pallas_call(
        _tc_mm_body,
        grid=(NRB,),
        in_specs=[pl.BlockSpec((RB, C_IN), lambda i: (i, 0)),
                  pl.BlockSpec((C_IN, C_HID), lambda i: (0, 0))],
        out_specs=pl.BlockSpec((RB, C_HID), lambda i: (i, 0)),
        out_shape=jax.ShapeDtypeStruct((N, C_HID), _f32),
    )(x, W1)


def _tc_scale_body(h_ref, p0_ref, p1_ref, hs_ref, dis_ref):
    deg = 1.0 + p0_ref[0, :, 0:1] + p1_ref[0, :, 0:1]
    dis = lax.rsqrt(deg)
    hs_ref[...] = h_ref[...] * dis
    dis_ref[...] = jnp.broadcast_to(dis, (RB, 8))


@jax.jit
def _tc_scale(h, degp):
    return pl.pallas_call(
        _tc_scale_body,
        grid=(NRB,),
        in_specs=[
            pl.BlockSpec((RB, C_HID), lambda i: (i, 0)),
            pl.BlockSpec((1, RB, DEGW), lambda i: (0, i, 0)),
            pl.BlockSpec((1, RB, DEGW), lambda i: (1, i, 0)),
        ],
        out_specs=[pl.BlockSpec((RB, C_HID), lambda i: (i, 0)),
                   pl.BlockSpec((RB, 8), lambda i: (i, 0))],
        out_shape=[jax.ShapeDtypeStruct((N, C_HID), _f32),
                   jax.ShapeDtypeStruct((N, 8), _f32)],
    )(h, degp, degp)


def _tc_b_body(a0_ref, a1_ref, a2_ref, a3_ref, hs_ref, dis_ref, b1_ref,
               w2_ref, hs2_ref):
    dis = dis_ref[:, 0:1]
    agg = jnp.concatenate(
        [a0_ref[0], a1_ref[0], a2_ref[0], a3_ref[0]], axis=1)
    h1 = dis * (agg + hs_ref[...]) + b1_ref[...]
    h1 = jnp.maximum(h1, 0.0)
    h2 = jnp.dot(h1, w2_ref[...], preferred_element_type=_f32)
    hs2_ref[...] = h2 * dis


@jax.jit
def _tc_b(acc, hs, dis, b1, W2):
    def blkspec(k):
        return pl.BlockSpec((1, RB, FS), lambda i, kk=k: (kk, i, 0))
    return pl.pallas_call(
        _tc_b_body,
        grid=(NRB,),
        in_specs=[blkspec(0), blkspec(1), blkspec(2), blkspec(3),
                  pl.BlockSpec((RB, C_HID), lambda i: (i, 0)),
                  pl.BlockSpec((RB, 8), lambda i: (i, 0)),
                  pl.BlockSpec((1, C_HID), lambda i: (0, 0)),
                  pl.BlockSpec((C_HID, C_OUT), lambda i: (0, 0))],
        out_specs=pl.BlockSpec((RB, C_OUT), lambda i: (i, 0)),
        out_shape=jax.ShapeDtypeStruct((N, C_OUT), _f32),
    )(acc, acc, acc, acc, hs, dis, b1, W2)


def _tc_c_body(a0_ref, a1_ref, hs2_ref, dis_ref, b2_ref, out_ref):
    dis = dis_ref[:, 0:1]
    agg = jnp.concatenate([a0_ref[0], a1_ref[0]], axis=1)
    out_ref[...] = dis * (agg + hs2_ref[...]) + b2_ref[...]


@jax.jit
def _tc_c(acc2, hs2, dis, b2):
    def blkspec(k):
        return pl.BlockSpec((1, RB, FS), lambda i, kk=k: (kk, i, 0))
    return pl.pallas_call(
        _tc_c_body,
        grid=(NRB,),
        in_specs=[blkspec(0), blkspec(1),
                  pl.BlockSpec((RB, C_OUT), lambda i: (i, 0)),
                  pl.BlockSpec((RB, 8), lambda i: (i, 0)),
                  pl.BlockSpec((1, C_OUT), lambda i: (0, 0))],
        out_specs=pl.BlockSpec((RB, C_OUT), lambda i: (i, 0)),
        out_shape=jax.ShapeDtypeStruct((N, C_OUT), _f32),
    )(acc2, acc2, hs2, dis, b2)


@jax.jit
def kernel(x, edge_index, W1, b1, W2, b2):
    row = edge_index[0].astype(jnp.int32).reshape(NS, E // NS)
    col = edge_index[1].astype(jnp.int32).reshape(NS, E // NS)
    padr = jnp.broadcast_to(jnp.arange(PADT, dtype=jnp.int32) % 512,
                            (NS, PADT))
    padc = jnp.broadcast_to(N + jnp.arange(PADT, dtype=jnp.int32) % 128,
                            (NS, PADT))
    row16 = jnp.concatenate([row, padr], axis=1).reshape(NS, NCH, CH)
    col16 = jnp.concatenate([col, padc], axis=1).reshape(NS, NCH, CH)

    degp = _deg_kernel(col16)
    h = _tc_mm(x, W1)
    hs, dis = _tc_scale(h, degp)
    acc = _scatter_hid(row16, col16, hs.reshape(N * 4, FS))
    hs2 = _tc_b(acc, hs, dis, b1.reshape(1, C_HID), W2)
    acc2 = _scatter_out(row16, col16, hs2.reshape(N * 2, FS))
    return _tc_c(acc2, hs2, dis, b2.reshape(1, C_OUT))


# speedup vs baseline: 1.0182x; 1.0182x over previous
import functools

import jax
import jax.numpy as jnp
from jax import lax
from jax.experimental import pallas as pl
from jax.experimental.pallas import tpu as pltpu
from jax.experimental.pallas import tpu_sc as plsc

N = 10000
E = 320000
C_IN = 128
C_HID = 128
C_OUT = 64
FS = 32

NC = 2
NS = 16
NW = NC * NS

CH = 128
GRP = 5

EPT = 20480
NCH = EPT // CH
NGRP = NCH // GRP
PADT = EPT - E // NS

NCHD = NCH // NC
NT = N + 128

BR = 624
XR = N - NS * BR
ZR = 208

DEGW = 16

_f32 = jnp.float32


def _mesh():
    return plsc.VectorSubcoreMesh(core_axis_name="c", subcore_axis_name="s")


_sc_params = pltpu.CompilerParams(use_tc_tiling_on_sc=False)


def _fill_zero(ref, rows, cols):
    zero16 = jnp.zeros((16,), _f32)

    def body(i, _):
        for j in range(cols // 16):
            ref[i, pl.ds(j * 16, 16)] = zero16
        return 0
    lax.fori_loop(0, rows, body, 0)


def _zero_and_barrier(zbuf_v, acc_sh, sid):
    for k in range(BR // ZR):
        pltpu.sync_copy(zbuf_v, acc_sh.at[pl.ds(sid * BR + k * ZR, ZR)])

    @pl.when(sid == NS - 1)
    def _():
        pltpu.sync_copy(zbuf_v.at[pl.ds(0, XR)],
                        acc_sh.at[pl.ds(NS * BR, XR)])
    plsc.subcore_barrier()


def _copy_out(acc_sh, out_slice, sid):
    pltpu.sync_copy(acc_sh.at[pl.ds(sid * BR, BR)],
                    out_slice.at[pl.ds(sid * BR, BR)])

    @pl.when(sid == NS - 1)
    def _():
        pltpu.sync_copy(acc_sh.at[pl.ds(NS * BR, XR)],
                        out_slice.at[pl.ds(NS * BR, XR)])


@functools.partial(
    pl.kernel,
    out_type=jax.ShapeDtypeStruct((NC, N, DEGW), _f32),
    mesh=_mesh(),
    compiler_params=_sc_params,
    scratch_types=[
        pltpu.VMEM((NCHD, CH), jnp.int32),
        pltpu.VMEM((CH, DEGW), _f32),
        pltpu.VMEM((ZR, DEGW), _f32),
        pltpu.VMEM_SHARED((NT, DEGW), _f32),
        pltpu.SemaphoreType.DMA,
    ],
)
def _deg_kernel(col_hbm, out_hbm, col_v, ones_v, zbuf_v, acc_sh, sem):
    cid = lax.axis_index("c")
    sid = lax.axis_index("s")

    pltpu.sync_copy(col_hbm.at[sid, pl.ds(cid * NCHD, NCHD)], col_v)

    one16 = jnp.ones((16,), _f32)

    def fill_ones(i, _):
        ones_v[i, :] = one16
        return 0
    lax.fori_loop(0, CH, fill_ones, 0)

    _fill_zero(zbuf_v, ZR, DEGW)
    _zero_and_barrier(zbuf_v, acc_sh, sid)

    def fire(g):
        for b in range(GRP):
            pltpu.async_copy(ones_v, acc_sh.at[col_v.at[g * GRP + b]],
                             sem, add=True)

    def drain(g):
        for b in range(GRP):
            pltpu.make_async_copy(ones_v, acc_sh.at[col_v.at[g * GRP + b]],
                                  sem).wait()

    fire(0)

    def body(g, _):
        @pl.when(g + 1 < NCHD // GRP)
        def _():
            fire(g + 1)
        drain(g)
        return 0
    lax.fori_loop(0, NCHD // GRP, body, 0)

    plsc.subcore_barrier()
    _copy_out(acc_sh, out_hbm.at[cid], sid)


NBUF = 12
LAG = 2


def _make_scatter(NBLK):
    @functools.partial(
        pl.kernel,
        out_type=jax.ShapeDtypeStruct((NBLK, N, FS), _f32),
        mesh=_mesh(),
        compiler_params=_sc_params,
        scratch_types=[
            pltpu.VMEM((NCH, CH), jnp.int32),
            pltpu.VMEM((NCH, CH), jnp.int32),
            pltpu.VMEM((NBUF, CH, FS), _f32),
            pltpu.VMEM((ZR, FS), _f32),
            pltpu.VMEM_SHARED((NT, FS), _f32),
            pltpu.SemaphoreType.DMA,
            pltpu.SemaphoreType.DMA,
        ],
    )
    def _scatter_kernel(row_hbm, col_hbm, tab_hbm, out_hbm,
                        idx_v, col_v, buf_v, zbuf_v, acc_sh, semg, sems):
        cid = lax.axis_index("c")
        sid = lax.axis_index("s")

        pltpu.sync_copy(row_hbm.at[sid], idx_v)
        pltpu.sync_copy(col_hbm.at[sid], col_v)
        _fill_zero(zbuf_v, ZR, FS)

        for p in range(NBLK // NC):
            blk = NC * p + cid
            step16 = (jnp.zeros((16,), jnp.int32) +
                      (blk if p == 0 else NC))

            def mkidx(j, _):
                for l in range(CH // 16):
                    sl = pl.ds(l * 16, 16)
                    v = idx_v[j, sl]
                    idx_v[j, sl] = (v * NBLK if p == 0 else v) + step16
                return 0
            lax.fori_loop(0, NCH, mkidx, 0)

            _zero_and_barrier(zbuf_v, acc_sh, sid)

            def fire_gather(j):
                slot = lax.rem(j, NBUF)
                pltpu.async_copy(tab_hbm.at[idx_v.at[j]],
                                 buf_v.at[slot], semg)

            def wait_scatter(j):
                slot = lax.rem(j, NBUF)
                pltpu.make_async_copy(buf_v.at[slot],
                                      acc_sh.at[col_v.at[j]], sems).wait()

            def prologue(j, _):
                fire_gather(j)
                return 0
            lax.fori_loop(0, NBUF, prologue, 0)

            def chunk(j, _):
                slot = lax.rem(j, NBUF)

                @pl.when(j >= LAG)
                def _():
                    wait_scatter(j - LAG)

                @pl.when(jnp.logical_and(j >= LAG, j + NBUF - LAG < NCH))
                def _():
                    fire_gather(j - LAG + NBUF)

                pltpu.make_async_copy(tab_hbm.at[idx_v.at[j]],
                                      buf_v.at[slot], semg).wait()
                pltpu.async_copy(buf_v.at[slot], acc_sh.at[col_v.at[j]],
                                 sems, add=True)
                return 0
            lax.fori_loop(0, NCH, chunk, 0)

            def epilogue(j, _):
                wait_scatter(j)
                return 0
            lax.fori_loop(NCH - LAG, NCH, epilogue, 0)

            plsc.subcore_barrier()
            _copy_out(acc_sh, out_hbm.at[blk], sid)
            plsc.subcore_barrier()

    return _scatter_kernel


_scatter_hid = _make_scatter(C_HID // FS)
_scatter_out = _make_scatter(C_OUT // FS)


RB = 400
NRB = N // RB


def _tc_a_body(x_ref, w1_ref, p0_ref, p1_ref, hs_ref, dis_ref):
    deg = 1.0 + p0_ref[0, :, 0:1] + p1_ref[0, :, 0:1]
    dis = lax.rsqrt(deg)
    h = jnp.dot(x_ref[...], w1_ref[...], preferred_element_type=_f32)
    hs_ref[...] = h * dis
    dis_ref[...] = jnp.broadcast_to(dis, (RB, 8))


@jax.jit
def _tc_a(x, W1, degp):
    return pl.pallas_call(
        _tc_a_body,
        grid=(NRB,),
        in_specs=[
            pl.BlockSpec((RB, C_IN), lambda i: (i, 0)),
            pl.BlockSpec((C_IN, C_HID), lambda i: (0, 0)),
            pl.BlockSpec((1, RB, DEGW), lambda i: (0, i, 0)),
            pl.BlockSpec((1, RB, DEGW), lambda i: (1, i, 0)),
        ],
        out_specs=[pl.BlockSpec((RB, C_HID), lambda i: (i, 0)),
                   pl.BlockSpec((RB, 8), lambda i: (i, 0))],
        out_shape=[jax.ShapeDtypeStruct((N, C_HID), _f32),
                   jax.ShapeDtypeStruct((N, 8), _f32)],
    )(x, W1, degp, degp)


def _tc_b_body(a0_ref, a1_ref, a2_ref, a3_ref, hs_ref, dis_ref, b1_ref,
               w2_ref, hs2_ref):
    dis = dis_ref[:, 0:1]
    agg = jnp.concatenate(
        [a0_ref[0], a1_ref[0], a2_ref[0], a3_ref[0]], axis=1)
    h1 = dis * (agg + hs_ref[...]) + b1_ref[...]
    h1 = jnp.maximum(h1, 0.0)
    h2 = jnp.dot(h1, w2_ref[...], preferred_element_type=_f32)
    hs2_ref[...] = h2 * dis


@jax.jit
def _tc_b(acc, hs, dis, b1, W2):
    def blkspec(k):
        return pl.BlockSpec((1, RB, FS), lambda i, kk=k: (kk, i, 0))
    return pl.pallas_call(
        _tc_b_body,
        grid=(NRB,),
        in_specs=[blkspec(0), blkspec(1), blkspec(2), blkspec(3),
                  pl.BlockSpec((RB, C_HID), lambda i: (i, 0)),
                  pl.BlockSpec((RB, 8), lambda i: (i, 0)),
                  pl.BlockSpec((1, C_HID), lambda i: (0, 0)),
                  pl.BlockSpec((C_HID, C_OUT), lambda i: (0, 0))],
        out_specs=pl.BlockSpec((RB, C_OUT), lambda i: (i, 0)),
        out_shape=jax.ShapeDtypeStruct((N, C_OUT), _f32),
    )(acc, acc, acc, acc, hs, dis, b1, W2)


def _tc_c_body(a0_ref, a1_ref, hs2_ref, dis_ref, b2_ref, out_ref):
    dis = dis_ref[:, 0:1]
    agg = jnp.concatenate([a0_ref[0], a1_ref[0]], axis=1)
    out_ref[...] = dis * (agg + hs2_ref[...]) + b2_ref[...]


@jax.jit
def _tc_c(acc2, hs2, dis, b2):
    def blkspec(k):
        return pl.BlockSpec((1, RB, FS), lambda i, kk=k: (kk, i, 0))
    return pl.pallas_call(
        _tc_c_body,
        grid=(NRB,),
        in_specs=[blkspec(0), blkspec(1),
                  pl.BlockSpec((RB, C_OUT), lambda i: (i, 0)),
                  pl.BlockSpec((RB, 8), lambda i: (i, 0)),
                  pl.BlockSpec((1, C_OUT), lambda i: (0, 0))],
        out_specs=pl.BlockSpec((RB, C_OUT), lambda i: (i, 0)),
        out_shape=jax.ShapeDtypeStruct((N, C_OUT), _f32),
    )(acc2, acc2, hs2, dis, b2)


@jax.jit
def kernel(x, edge_index, W1, b1, W2, b2):
    row = edge_index[0].astype(jnp.int32).reshape(NS, E // NS)
    col = edge_index[1].astype(jnp.int32).reshape(NS, E // NS)
    padr = jnp.broadcast_to(jnp.arange(PADT, dtype=jnp.int32) % 512,
                            (NS, PADT))
    padc = jnp.broadcast_to(N + jnp.arange(PADT, dtype=jnp.int32) % 128,
                            (NS, PADT))
    row16 = jnp.concatenate([row, padr], axis=1).reshape(NS, NCH, CH)
    col16 = jnp.concatenate([col, padc], axis=1).reshape(NS, NCH, CH)

    degp = _deg_kernel(col16)
    hs, dis = _tc_a(x, W1, degp)
    acc = _scatter_hid(row16, col16, hs.reshape(N * 4, FS))
    hs2 = _tc_b(acc, hs, dis, b1.reshape(1, C_HID), W2)
    acc2 = _scatter_out(row16, col16, hs2.reshape(N * 2, FS))
    return _tc_c(acc2, hs2, dis, b2.reshape(1, C_OUT))
